# Initial kernel scaffold; baseline (speedup 1.0000x reference)
import jax
import jax.numpy as jnp
from jax.experimental import pallas as pl


def kernel(logits):
    raise NotImplementedError("write your pallas kernel here")


# speedup vs baseline: 27.2388x; 27.2388x over previous
import jax
import jax.numpy as jnp
from jax.experimental import pallas as pl
from jax.experimental.pallas import tpu as pltpu

_K = 64
_INV_TAU = 10.0
_R = 32
_W = 1_000_000
_NC = 1000
_CW = 1000
_SLAB = 125
_MIN32 = -2147483648


def _monotone_key(x):
    b = jax.lax.bitcast_convert_type(x, jnp.int32)
    return jnp.where(b >= 0, b, jnp.int32(_MIN32) - b)


def _row_body(x_ref, o_ref, cm_ref, cand_ref, key_ref):
    for j in range(_NC // _SLAB):
        sl = x_ref[pl.ds(j * _SLAB, _SLAB), :]
        cm_ref[0, pl.ds(j * _SLAB, _SLAB)] = jnp.max(sl, axis=1)

    iota = jax.lax.broadcasted_iota(jnp.int32, (1, _NC), 1)

    def gather_body(s, cm):
        m = jnp.max(cm)
        idx = jnp.min(jnp.where(cm == m, iota, jnp.int32(_NC)))
        cand_ref[pl.ds(s, 1), :] = x_ref[pl.ds(idx, 1), :]
        return jnp.where(iota == idx, jnp.float32(-jnp.inf), cm)

    jax.lax.fori_loop(0, _K, gather_body, cm_ref[0:1, :])

    key_ref[...] = _monotone_key(cand_ref[...])

    def count_ge(t):
        return jnp.sum((key_ref[...] >= t).astype(jnp.int32))

    t_key = jnp.where(count_ge(jnp.int32(0)) >= _K, jnp.int32(0),
                      jnp.int32(_MIN32))

    def bit_body(i, t):
        cand_t = t + (jnp.int32(1) << (jnp.int32(30) - i))
        return jnp.where(count_ge(cand_t) >= _K, cand_t, t)

    t_key = jax.lax.fori_loop(0, 31, bit_body, t_key)
    v_bits = jnp.where(t_key >= 0, t_key, jnp.int32(_MIN32) - t_key)
    v = jax.lax.bitcast_convert_type(v_bits, jnp.float32)

    for j in range(_NC // _SLAB):
        xs = x_ref[pl.ds(j * _SLAB, _SLAB), :]
        z = (v - xs) * jnp.float32(_INV_TAU)
        o_ref[pl.ds(j * _SLAB, _SLAB), :] = xs / (1.0 + jnp.exp(z))


def kernel(logits):
    x = logits.reshape(_R * _NC, _CW)
    out = pl.pallas_call(
        _row_body,
        grid=(_R,),
        in_specs=[pl.BlockSpec((_NC, _CW), lambda i: (i, 0))],
        out_specs=pl.BlockSpec((_NC, _CW), lambda i: (i, 0)),
        out_shape=jax.ShapeDtypeStruct((_R * _NC, _CW), jnp.float32),
        scratch_shapes=[
            pltpu.VMEM((1, _NC), jnp.float32),
            pltpu.VMEM((_K, _CW), jnp.float32),
            pltpu.VMEM((_K, _CW), jnp.int32),
        ],
    )(x)
    return out.reshape(_R, _W)


# speedup vs baseline: 50.8651x; 1.8674x over previous
import jax
import jax.numpy as jnp
from jax.experimental import pallas as pl
from jax.experimental.pallas import tpu as pltpu

_K = 64
_INV_TAU = 10.0
_R = 32
_W = 1_000_000
_NC = 1000
_CW = 1000
_SLAB = 125
_MIN32 = -2147483648


def _monotone_key(x):
    b = jax.lax.bitcast_convert_type(x, jnp.int32)
    return jnp.where(b >= 0, b, jnp.int32(_MIN32) - b)


def _chunkmax_body(x_ref, cm_ref):
    for j in range(_NC // _SLAB):
        sl = x_ref[pl.ds(j * _SLAB, _SLAB), :]
        mx = jnp.max(sl, axis=1).reshape(1, _SLAB)
        pad = jnp.full((1, 128 - _SLAB), -jnp.inf, jnp.float32)
        cm_ref[0, pl.ds(j, 1), :] = jnp.concatenate([mx, pad], axis=1)


_NP = 1024


def _select_body(cm_ref, perm_ref):
    p_iota = jax.lax.broadcasted_iota(jnp.int32, (_R, _NP), 1)
    kk = jax.lax.rem(p_iota, jnp.int32(128))
    valid = kk < _SLAB
    cidx = (p_iota // 128) * _SLAB + kk
    keys = jnp.where(valid, _monotone_key(cm_ref[...]), jnp.int32(_MIN32))
    cnt = jnp.sum((keys >= 0).astype(jnp.int32), axis=1, keepdims=True)
    t = jnp.where(cnt >= _K, jnp.int32(0), jnp.int32(_MIN32))
    for b in range(30, -1, -1):
        cand_t = t + jnp.int32(1 << b)
        cnt = jnp.sum((keys >= cand_t).astype(jnp.int32), axis=1,
                      keepdims=True)
        t = jnp.where(cnt >= _K, cand_t, t)
    tri_r = jax.lax.broadcasted_iota(jnp.int32, (_NP, _NP), 0)
    tri_c = jax.lax.broadcasted_iota(jnp.int32, (_NP, _NP), 1)
    le = (tri_r <= tri_c).astype(jnp.float32)
    above = (keys > t)
    q = jnp.sum(above.astype(jnp.float32), axis=1, keepdims=True)
    tie = (keys == t)
    tie_rank = jnp.dot(tie.astype(jnp.float32), le,
                       preferred_element_type=jnp.float32)
    sel = above | (tie & (tie_rank <= (_K - q)))
    rank = jnp.dot(sel.astype(jnp.float32), le,
                   preferred_element_type=jnp.float32).astype(jnp.int32) - 1
    picked = jnp.where(sel, rank, jnp.int32(-1))
    for s in range(_K):
        perm_ref[:, pl.ds(s, 1)] = jnp.sum(
            jnp.where(picked == s, cidx, 0), axis=1, keepdims=True)


def _gate_body(perm_ref, x_ref, o_ref, cand_ref, key_ref):
    i = pl.program_id(0)
    for s in range(_K):
        idx = perm_ref[i, s]
        cand_ref[pl.ds(s, 1), :] = x_ref[pl.ds(idx, 1), :]
    key_ref[...] = _monotone_key(cand_ref[...])

    def count_ge(tt):
        return jnp.sum((key_ref[...] >= tt).astype(jnp.int32), axis=(0, 1),
                       keepdims=True)

    t = jnp.where(count_ge(jnp.int32(0)) >= _K, jnp.int32(0),
                  jnp.int32(_MIN32))
    for b in range(30, -1, -1):
        cand_t = t + jnp.int32(1 << b)
        t = jnp.where(count_ge(cand_t) >= _K, cand_t, t)
    v_bits = jnp.where(t >= 0, t, jnp.int32(_MIN32) - t)
    v = jax.lax.bitcast_convert_type(v_bits, jnp.float32)

    for j in range(_NC // _SLAB):
        xs = x_ref[pl.ds(j * _SLAB, _SLAB), :]
        z = (v - xs) * jnp.float32(_INV_TAU)
        o_ref[pl.ds(j * _SLAB, _SLAB), :] = xs / (1.0 + jnp.exp(z))


def kernel(logits):
    x = logits.reshape(_R * _NC, _CW)
    cm = pl.pallas_call(
        _chunkmax_body,
        grid=(_R,),
        in_specs=[pl.BlockSpec((_NC, _CW), lambda i: (i, 0))],
        out_specs=pl.BlockSpec((1, 8, 128), lambda i: (i, 0, 0)),
        out_shape=jax.ShapeDtypeStruct((_R, 8, 128), jnp.float32),
    )(x)
    perm = pl.pallas_call(
        _select_body,
        out_shape=jax.ShapeDtypeStruct((_R, _K), jnp.int32),
    )(cm.reshape(_R, _NP))
    out = pl.pallas_call(
        _gate_body,
        grid=(_R,),
        in_specs=[
            pl.BlockSpec(memory_space=pltpu.SMEM),
            pl.BlockSpec((_NC, _CW), lambda i: (i, 0)),
        ],
        out_specs=pl.BlockSpec((_NC, _CW), lambda i: (i, 0)),
        out_shape=jax.ShapeDtypeStruct((_R * _NC, _CW), jnp.float32),
        scratch_shapes=[
            pltpu.VMEM((_K, _CW), jnp.float32),
            pltpu.VMEM((_K, _CW), jnp.int32),
        ],
    )(perm, x)
    return out.reshape(_R, _W)
